# initial kernel scaffold (unmeasured)
import jax
import jax.numpy as jnp
from jax import lax
from jax.experimental import pallas as pl
from jax.experimental.pallas import tpu as pltpu

N_DEV = 8


def kernel(x, w_mat):
    m_per, k = x.shape
    n_per = w_mat.shape[1]

    xb = x.astype(jnp.bfloat16)
    wb = w_mat.astype(jnp.bfloat16)

    def body(x_hbm, w_ref, out_ref, comm_ref, copy_sem, send_sems, recv_sems):
        my = lax.axis_index("i")
        left = lax.rem(my + N_DEV - 1, N_DEV)
        right = lax.rem(my + 1, N_DEV)

        cp = pltpu.make_async_copy(x_hbm, comm_ref.at[0], copy_sem)
        cp.start()
        cp.wait()

        barrier_sem = pltpu.get_barrier_semaphore()
        for nbr in (left, right):
            pl.semaphore_signal(
                barrier_sem, inc=1,
                device_id=(nbr,), device_id_type=pl.DeviceIdType.MESH,
            )
        pl.semaphore_wait(barrier_sem, 2)

        out_ref[pl.ds(my * m_per, m_per), :] = jnp.dot(
            comm_ref[0], w_ref[...], preferred_element_type=jnp.float32
        )

        for h in range(N_DEV - 1):
            s = h % 2
            r = (h + 1) % 2
            rdma = pltpu.make_async_remote_copy(
                src_ref=comm_ref.at[s],
                dst_ref=comm_ref.at[r],
                send_sem=send_sems.at[s],
                recv_sem=recv_sems.at[r],
                device_id=(right,),
                device_id_type=pl.DeviceIdType.MESH,
            )
            rdma.start()
            rdma.wait()

            origin = lax.rem(my - (h + 1) + N_DEV, N_DEV)
            out_ref[pl.ds(origin * m_per, m_per), :] = jnp.dot(
                comm_ref[r], w_ref[...], preferred_element_type=jnp.float32
            )

    return pl.pallas_call(
        body,
        out_shape=jax.ShapeDtypeStruct((N_DEV * m_per, n_per), jnp.float32),
        in_specs=[
            pl.BlockSpec(memory_space=pltpu.ANY),
            pl.BlockSpec(memory_space=pltpu.VMEM),
        ],
        out_specs=pl.BlockSpec(memory_space=pltpu.VMEM),
        scratch_shapes=[
            pltpu.VMEM((2, m_per, k), jnp.bfloat16),
            pltpu.SemaphoreType.DMA,
            pltpu.SemaphoreType.DMA((2,)),
            pltpu.SemaphoreType.DMA((2,)),
        ],
        compiler_params=pltpu.CompilerParams(collective_id=0),
    )(xb, wb)


# baseline (device time: 1405446 ns/iter reference)
import jax
import jax.numpy as jnp
from jax import lax
from jax.experimental import pallas as pl
from jax.experimental.pallas import tpu as pltpu

jax.config.update("jax_compilation_cache_dir", "/tmp/jax_comp_cache")
jax.config.update("jax_persistent_cache_min_compile_time_secs", 0.5)

N_DEV = 8


def kernel(x, w_mat):
    m_per, k = x.shape
    n_per = w_mat.shape[1]

    xb = x.astype(jnp.bfloat16)
    wb = w_mat.astype(jnp.bfloat16)

    def body(x_hbm, w_ref, out_ref, comm_ref, copy_sem, send_sems, recv_sems):
        my = lax.axis_index("i")
        left = lax.rem(my + N_DEV - 1, N_DEV)
        right = lax.rem(my + 1, N_DEV)

        cp = pltpu.make_async_copy(x_hbm, comm_ref.at[0], copy_sem)
        cp.start()
        cp.wait()

        barrier_sem = pltpu.get_barrier_semaphore()
        for nbr in (left, right):
            pl.semaphore_signal(
                barrier_sem, inc=1,
                device_id=(nbr,), device_id_type=pl.DeviceIdType.MESH,
            )
        pl.semaphore_wait(barrier_sem, 2)

        out_ref[pl.ds(my * m_per, m_per), :] = jnp.dot(
            comm_ref[0], w_ref[...], preferred_element_type=jnp.float32
        )

        def hop(h, carry):
            s = lax.rem(h, 2)
            r = lax.rem(h + 1, 2)
            rdma = pltpu.make_async_remote_copy(
                src_ref=comm_ref.at[s],
                dst_ref=comm_ref.at[r],
                send_sem=send_sems.at[s],
                recv_sem=recv_sems.at[r],
                device_id=(right,),
                device_id_type=pl.DeviceIdType.MESH,
            )
            rdma.start()
            rdma.wait()

            origin = lax.rem(my - h - 1 + N_DEV, N_DEV)
            out_ref[pl.ds(origin * m_per, m_per), :] = jnp.dot(
                comm_ref[r], w_ref[...], preferred_element_type=jnp.float32
            )
            return carry

        lax.fori_loop(0, N_DEV - 1, hop, 0)

    return pl.pallas_call(
        body,
        out_shape=jax.ShapeDtypeStruct((N_DEV * m_per, n_per), jnp.float32),
        in_specs=[
            pl.BlockSpec(memory_space=pltpu.MemorySpace.HBM),
            pl.BlockSpec(memory_space=pltpu.VMEM),
        ],
        out_specs=pl.BlockSpec(memory_space=pltpu.VMEM),
        scratch_shapes=[
            pltpu.VMEM((2, m_per, k), jnp.bfloat16),
            pltpu.SemaphoreType.DMA,
            pltpu.SemaphoreType.DMA((2,)),
            pltpu.SemaphoreType.DMA((2,)),
        ],
        compiler_params=pltpu.CompilerParams(
            collective_id=0,
            vmem_limit_bytes=64 * 1024 * 1024,
        ),
    )(xb, wb)


# device time: 711794 ns/iter; 1.9745x vs baseline; 1.9745x over previous
import jax
import jax.numpy as jnp
from jax import lax
from jax.experimental import pallas as pl
from jax.experimental.pallas import tpu as pltpu

jax.config.update("jax_compilation_cache_dir", "/tmp/jax_comp_cache")
jax.config.update("jax_persistent_cache_min_compile_time_secs", 0.5)

N_DEV = 8


def kernel(x, w_mat):
    m_per, k = x.shape
    n_per = w_mat.shape[1]
    m_half = m_per // 2

    xb = x.astype(jnp.bfloat16)
    wb = w_mat.astype(jnp.bfloat16)

    def body(
        x_hbm, w_ref, out_ref,
        cw_ref, ccw_ref,
        copy_sems,
        cw_send, cw_recv, ccw_send, ccw_recv,
        credit_sems,
    ):
        my = lax.axis_index("i")
        left = lax.rem(my + N_DEV - 1, N_DEV)
        right = lax.rem(my + 1, N_DEV)

        cp_top = pltpu.make_async_copy(
            x_hbm.at[pl.ds(0, m_half), :], cw_ref.at[0], copy_sems.at[0]
        )
        cp_bot = pltpu.make_async_copy(
            x_hbm.at[pl.ds(m_half, m_half), :], ccw_ref.at[0], copy_sems.at[1]
        )
        cp_top.start()
        cp_bot.start()
        cp_top.wait()
        cp_bot.wait()

        barrier_sem = pltpu.get_barrier_semaphore()
        for nbr in (left, right):
            pl.semaphore_signal(
                barrier_sem, inc=1,
                device_id=(nbr,), device_id_type=pl.DeviceIdType.MESH,
            )
        pl.semaphore_wait(barrier_sem, 2)

        def compute_slot(slot, h):
            cw_origin = lax.rem(my - h + N_DEV, N_DEV)
            ccw_origin = lax.rem(my + h, N_DEV)
            out_ref[pl.ds(cw_origin * m_per, m_half), :] = jnp.dot(
                cw_ref[slot], w_ref[...], preferred_element_type=jnp.float32
            )
            out_ref[pl.ds(ccw_origin * m_per + m_half, m_half), :] = jnp.dot(
                ccw_ref[slot], w_ref[...], preferred_element_type=jnp.float32
            )

        def hop(h, carry):
            s = lax.rem(h, 2)
            r = lax.rem(h + 1, 2)

            @pl.when(h >= 1)
            def _():
                pl.semaphore_wait(credit_sems.at[0], 1)
                pl.semaphore_wait(credit_sems.at[1], 1)

            cw = pltpu.make_async_remote_copy(
                src_ref=cw_ref.at[s],
                dst_ref=cw_ref.at[r],
                send_sem=cw_send.at[s],
                recv_sem=cw_recv.at[r],
                device_id=(right,),
                device_id_type=pl.DeviceIdType.MESH,
            )
            ccw = pltpu.make_async_remote_copy(
                src_ref=ccw_ref.at[s],
                dst_ref=ccw_ref.at[r],
                send_sem=ccw_send.at[s],
                recv_sem=ccw_recv.at[r],
                device_id=(left,),
                device_id_type=pl.DeviceIdType.MESH,
            )
            cw.start()
            ccw.start()

            compute_slot(s, h)

            cw.wait_send()
            ccw.wait_send()

            @pl.when(h < N_DEV - 2)
            def _():
                pl.semaphore_signal(
                    credit_sems.at[0], inc=1,
                    device_id=(left,), device_id_type=pl.DeviceIdType.MESH,
                )
                pl.semaphore_signal(
                    credit_sems.at[1], inc=1,
                    device_id=(right,), device_id_type=pl.DeviceIdType.MESH,
                )

            cw.wait_recv()
            ccw.wait_recv()
            return carry

        lax.fori_loop(0, N_DEV - 1, hop, 0)

        compute_slot(1, N_DEV - 1)

    return pl.pallas_call(
        body,
        out_shape=jax.ShapeDtypeStruct((N_DEV * m_per, n_per), jnp.float32),
        in_specs=[
            pl.BlockSpec(memory_space=pltpu.MemorySpace.HBM),
            pl.BlockSpec(memory_space=pltpu.VMEM),
        ],
        out_specs=pl.BlockSpec(memory_space=pltpu.VMEM),
        scratch_shapes=[
            pltpu.VMEM((2, m_half, k), jnp.bfloat16),
            pltpu.VMEM((2, m_half, k), jnp.bfloat16),
            pltpu.SemaphoreType.DMA((2,)),
            pltpu.SemaphoreType.DMA((2,)),
            pltpu.SemaphoreType.DMA((2,)),
            pltpu.SemaphoreType.DMA((2,)),
            pltpu.SemaphoreType.DMA((2,)),
            pltpu.SemaphoreType.REGULAR((2,)),
        ],
        compiler_params=pltpu.CompilerParams(
            collective_id=0,
            vmem_limit_bytes=64 * 1024 * 1024,
        ),
    )(xb, wb)


# device time: 695631 ns/iter; 2.0204x vs baseline; 1.0232x over previous
import jax
import jax.numpy as jnp
from jax import lax
from jax.experimental import pallas as pl
from jax.experimental.pallas import tpu as pltpu

jax.config.update("jax_compilation_cache_dir", "/tmp/jax_comp_cache")
jax.config.update("jax_persistent_cache_min_compile_time_secs", 0.5)

N_DEV = 8
N_STREAMS = 4


def kernel(x, w_mat):
    m_per, k = x.shape
    n_per = w_mat.shape[1]
    m_q = m_per // 4

    xb = x.astype(jnp.bfloat16)
    wb = w_mat.astype(jnp.bfloat16)

    def body(
        x_hbm, w_ref, out_ref,
        buf0, buf1, buf2, buf3,
        copy_sems, send_sems, recv_sems, credit_sems,
    ):
        my = lax.axis_index("i")
        left = lax.rem(my + N_DEV - 1, N_DEV)
        right = lax.rem(my + 1, N_DEV)

        bufs = [buf0, buf1, buf2, buf3]
        row_off = [0, 512, 256, 768]
        sign = [1, -1, 1, -1]
        target = [right, left, right, left]
        grant_to = [left, right, left, right]

        def descr(st, src_slot, dst_slot):
            return pltpu.make_async_remote_copy(
                src_ref=bufs[st].at[src_slot],
                dst_ref=bufs[st].at[dst_slot],
                send_sem=send_sems.at[st, src_slot],
                recv_sem=recv_sems.at[st, dst_slot],
                device_id=(target[st],),
                device_id_type=pl.DeviceIdType.MESH,
            )

        def compute(st, slot, h):
            origin = lax.rem(my - sign[st] * (h + 1) + (h + 1) * N_DEV, N_DEV)
            out_ref[pl.ds(origin * m_per + row_off[st], m_q), :] = jnp.dot(
                bufs[st][slot], w_ref[...], preferred_element_type=jnp.float32
            )

        cps = [
            pltpu.make_async_copy(
                x_hbm.at[pl.ds(row_off[st], m_q), :],
                bufs[st].at[0],
                copy_sems.at[st],
            )
            for st in range(N_STREAMS)
        ]
        for cp in cps:
            cp.start()
        for cp in cps:
            cp.wait()

        barrier_sem = pltpu.get_barrier_semaphore()
        for nbr in (left, right):
            pl.semaphore_signal(
                barrier_sem, inc=1,
                device_id=(nbr,), device_id_type=pl.DeviceIdType.MESH,
            )
        pl.semaphore_wait(barrier_sem, 2)

        for st in range(N_STREAMS):
            descr(st, 0, 1).start()
        for st in range(N_STREAMS):
            compute(st, 0, -1)

        def hop(h, carry):
            s = lax.rem(h, 2)
            r = lax.rem(h + 1, 2)
            for st_pair in ((0, 1), (2, 3)):
                for st in st_pair:
                    d = descr(st, s, r)
                    d.wait_recv()
                    d.wait_send()

                    @pl.when(h < N_DEV - 2)
                    def _():
                        pl.semaphore_signal(
                            credit_sems.at[st], inc=1,
                            device_id=(grant_to[st],),
                            device_id_type=pl.DeviceIdType.MESH,
                        )

                    @pl.when(h < N_DEV - 2)
                    def _():
                        pl.semaphore_wait(credit_sems.at[st], 1)
                        descr(st, r, s).start()

                for st in st_pair:
                    compute(st, r, h)
            return carry

        lax.fori_loop(0, N_DEV - 1, hop, 0)

    return pl.pallas_call(
        body,
        out_shape=jax.ShapeDtypeStruct((N_DEV * m_per, n_per), jnp.float32),
        in_specs=[
            pl.BlockSpec(memory_space=pltpu.MemorySpace.HBM),
            pl.BlockSpec(memory_space=pltpu.VMEM),
        ],
        out_specs=pl.BlockSpec(memory_space=pltpu.VMEM),
        scratch_shapes=[
            pltpu.VMEM((2, m_q, k), jnp.bfloat16),
            pltpu.VMEM((2, m_q, k), jnp.bfloat16),
            pltpu.VMEM((2, m_q, k), jnp.bfloat16),
            pltpu.VMEM((2, m_q, k), jnp.bfloat16),
            pltpu.SemaphoreType.DMA((N_STREAMS,)),
            pltpu.SemaphoreType.DMA((N_STREAMS, 2)),
            pltpu.SemaphoreType.DMA((N_STREAMS, 2)),
            pltpu.SemaphoreType.REGULAR((N_STREAMS,)),
        ],
        compiler_params=pltpu.CompilerParams(
            collective_id=0,
            vmem_limit_bytes=64 * 1024 * 1024,
        ),
    )(xb, wb)
